# baseline (speedup 1.0000x reference)
import functools

import jax
import jax.numpy as jnp
from jax import lax
from jax.experimental import pallas as pl
from jax.experimental.pallas import tpu as pltpu
from jax.experimental.pallas import tpu_sc as plsc

B, D, T = 8, 256, 2048
CODE_DIM, K = 64, 1024
TT = 1024
TT2 = 2048


def _prep_body(wout_ref, wconv_ref, bout_ref, bconv_ref, cb_ref,
               m2_ref, cbp_ref, cbn_ref):
    wo = wout_ref[...]
    wc = wconv_ref[...]
    m2 = lax.dot_general(wc, wo, (((1,), (1,)), ((), ())),
                         preferred_element_type=jnp.float32)
    bf = lax.dot_general(bout_ref[...], wc, (((1,), (1,)), ((), ())),
                         preferred_element_type=jnp.float32) + bconv_ref[...]
    bf_col = jnp.transpose(bf, (1, 0))
    m2_ref[...] = jnp.concatenate(
        [m2, bf_col, jnp.zeros((D, 127 - CODE_DIM), jnp.float32)], axis=1)
    cb = cb_ref[...]
    cbp_ref[...] = jnp.concatenate(
        [cb, jnp.ones((K, 1), jnp.float32),
         jnp.zeros((K, 127 - CODE_DIM), jnp.float32)], axis=1)
    cn = jnp.sqrt(jnp.sum(cb * cb, axis=1, keepdims=True))
    cbn_ref[...] = cb / jnp.maximum(cn, 1e-12)


def _argmax_body(h_ref, win_ref, bin_ref, cbn_ref, idx_ref):
    xb = h_ref[0]
    w = win_ref[...]
    cbn = cbn_ref[...]
    ec = lax.dot_general(xb, w, (((0,), (0,)), ((), ())),
                         preferred_element_type=jnp.float32)
    ec = ec + bin_ref[...]
    n = jnp.sqrt(jnp.sum(ec * ec, axis=1, keepdims=True))
    en = ec / jnp.maximum(n, 1e-8)
    logits = lax.dot_general(en, cbn, (((1,), (1,)), ((), ())),
                             preferred_element_type=jnp.float32)
    m = jnp.max(logits, axis=1, keepdims=True)
    unn = jnp.exp(logits - m)
    s = jnp.sum(unn, axis=1, keepdims=True)
    probs = unn / s
    idx = jnp.argmax(probs, axis=1).astype(jnp.int32)
    idx_ref[...] = idx.reshape(1, 1, TT)


def _out_body(c1a_ref, c1b_ref, m2_ref, out_ref):
    b = pl.program_id(0)
    c1 = jnp.where(b < B // 2, c1a_ref[0], c1b_ref[0])
    m2 = m2_ref[...]
    out_ref[0] = lax.dot_general(m2, c1, (((1,), (1,)), ((), ())),
                                 preferred_element_type=jnp.float32)


def _make_sc_gather(n_idx, d):
    info = plsc.get_sparse_core_info()
    nc, ns = info.num_cores, info.num_subcores
    nw = nc * ns
    per_w = n_idx // nw
    chunk = 128
    n_chunks = per_w // chunk
    mesh = plsc.VectorSubcoreMesh(core_axis_name="c", subcore_axis_name="s")

    @functools.partial(
        pl.kernel, mesh=mesh,
        out_type=jax.ShapeDtypeStruct((n_idx, d), jnp.float32),
        scratch_types=[
            pltpu.VMEM((n_chunks, chunk), jnp.int32),
            pltpu.VMEM((per_w, d), jnp.float32),
            pltpu.SemaphoreType.DMA,
        ],
    )
    def gather_k(idx_hbm, table_hbm, out_hbm, idx_v, rows_v, sem):
        wid = lax.axis_index("s") * nc + lax.axis_index("c")
        pltpu.sync_copy(idx_hbm.at[wid], idx_v)
        handles = [
            pltpu.async_copy(table_hbm.at[idx_v.at[j]],
                             rows_v.at[pl.ds(j * chunk, chunk)], sem)
            for j in range(n_chunks)
        ]
        for h in handles:
            h.wait()
        pltpu.sync_copy(rows_v, out_hbm.at[pl.ds(wid * per_w, per_w)])

    def run(idx_flat, table):
        nonlocal nw, n_chunks
        idx3 = idx_flat.reshape(nw, n_chunks, chunk)
        return gather_k(idx3, table)

    return run


def kernel(h_input, W_in, b_in, W_out, b_out, codebook, W_conv, b_conv):
    m2, cb_pad, cbn = pl.pallas_call(
        _prep_body,
        out_shape=[
            jax.ShapeDtypeStruct((D, 128), jnp.float32),
            jax.ShapeDtypeStruct((K, 128), jnp.float32),
            jax.ShapeDtypeStruct((K, CODE_DIM), jnp.float32),
        ],
    )(W_out, W_conv, b_out.reshape(1, D), b_conv.reshape(1, D), codebook)

    bh = B // 2

    def _argmax_half(off):
        return pl.pallas_call(
            _argmax_body,
            grid=(bh, T // TT),
            in_specs=[
                pl.BlockSpec((1, D, TT), lambda b, t: (b + off, 0, t)),
                pl.BlockSpec((D, CODE_DIM), lambda b, t: (0, 0)),
                pl.BlockSpec((1, CODE_DIM), lambda b, t: (0, 0)),
                pl.BlockSpec((K, CODE_DIM), lambda b, t: (0, 0)),
            ],
            out_specs=pl.BlockSpec((1, 1, TT), lambda b, t: (b, 0, t)),
            out_shape=jax.ShapeDtypeStruct((bh, 1, T), jnp.int32),
            compiler_params=pltpu.CompilerParams(
                dimension_semantics=("parallel", "parallel")),
        )(h_input, W_in, b_in.reshape(1, CODE_DIM), cbn)

    idxa = _argmax_half(0)
    idxb = _argmax_half(bh)

    gather = _make_sc_gather(bh * T, 128)
    c1a = gather(idxa.reshape(bh * T), cb_pad)
    c1b = gather(idxb.reshape(bh * T), cb_pad)

    out = pl.pallas_call(
        _out_body,
        grid=(B, T // TT2),
        in_specs=[
            pl.BlockSpec((1, TT2, 128),
                         lambda b, t: (jnp.minimum(b, bh - 1), t, 0)),
            pl.BlockSpec((1, TT2, 128),
                         lambda b, t: (jnp.maximum(b - bh, 0), t, 0)),
            pl.BlockSpec((D, 128), lambda b, t: (0, 0)),
        ],
        out_specs=pl.BlockSpec((1, D, TT2), lambda b, t: (b, 0, t)),
        out_shape=jax.ShapeDtypeStruct((B, D, T), jnp.float32),
    )(c1a.reshape(bh, T, 128), c1b.reshape(bh, T, 128), m2)
    return out


# speedup vs baseline: 1.0253x; 1.0253x over previous
import functools

import jax
import jax.numpy as jnp
from jax import lax
from jax.experimental import pallas as pl
from jax.experimental.pallas import tpu as pltpu
from jax.experimental.pallas import tpu_sc as plsc

B, D, T = 8, 256, 2048
CODE_DIM, K = 64, 1024
TT = 1024
TT2 = 2048


def _prep_body(wout_ref, wconv_ref, bout_ref, bconv_ref, cb_ref,
               m2_ref, cbp_ref, cbn_ref):
    wo = wout_ref[...]
    wc = wconv_ref[...]
    m2 = lax.dot_general(wc, wo, (((1,), (1,)), ((), ())),
                         preferred_element_type=jnp.float32)
    bf = lax.dot_general(bout_ref[...], wc, (((1,), (1,)), ((), ())),
                         preferred_element_type=jnp.float32) + bconv_ref[...]
    bf_col = jnp.transpose(bf, (1, 0))
    m2_ref[...] = jnp.concatenate(
        [m2, bf_col, jnp.zeros((D, 127 - CODE_DIM), jnp.float32)], axis=1)
    cb = cb_ref[...]
    cbp_ref[...] = jnp.concatenate(
        [cb, jnp.ones((K, 1), jnp.float32),
         jnp.zeros((K, 127 - CODE_DIM), jnp.float32)], axis=1)
    cn = jnp.sqrt(jnp.sum(cb * cb, axis=1, keepdims=True))
    cbn_ref[...] = cb / jnp.maximum(cn, 1e-12)


def _argmax_body(h_ref, win_ref, bin_ref, cbn_ref, idx_ref):
    xb = h_ref[0]
    w = win_ref[...]
    cbn = cbn_ref[...]
    ec = lax.dot_general(xb, w, (((0,), (0,)), ((), ())),
                         preferred_element_type=jnp.float32)
    ec = ec + bin_ref[...]
    n = jnp.sqrt(jnp.sum(ec * ec, axis=1, keepdims=True))
    en = ec / jnp.maximum(n, 1e-8)
    logits = lax.dot_general(en, cbn, (((1,), (1,)), ((), ())),
                             preferred_element_type=jnp.float32)
    m = jnp.max(logits, axis=1, keepdims=True)
    unn = jnp.exp(logits - m)
    s = jnp.sum(unn, axis=1, keepdims=True)
    probs = unn / s
    idx = jnp.argmax(probs, axis=1).astype(jnp.int32)
    idx_ref[...] = idx.reshape(1, 1, TT)


def _out_body(c1_ref, m2_ref, out_ref):
    c1 = c1_ref[0]
    m2 = m2_ref[...]
    out_ref[0] = lax.dot_general(m2, c1, (((1,), (1,)), ((), ())),
                                 preferred_element_type=jnp.float32)


def _out_body_acc(c1_ref, m2_ref, prev_ref, out_ref):
    del prev_ref
    c1 = c1_ref[0]
    m2 = m2_ref[...]
    out_ref[0] = lax.dot_general(m2, c1, (((1,), (1,)), ((), ())),
                                 preferred_element_type=jnp.float32)


def _make_sc_gather(n_idx, d):
    info = plsc.get_sparse_core_info()
    nc, ns = info.num_cores, info.num_subcores
    nw = nc * ns
    per_w = n_idx // nw
    chunk = 128
    n_chunks = per_w // chunk
    mesh = plsc.VectorSubcoreMesh(core_axis_name="c", subcore_axis_name="s")

    @functools.partial(
        pl.kernel, mesh=mesh,
        out_type=jax.ShapeDtypeStruct((n_idx, d), jnp.float32),
        scratch_types=[
            pltpu.VMEM((n_chunks, chunk), jnp.int32),
            pltpu.VMEM((per_w, d), jnp.float32),
            pltpu.SemaphoreType.DMA,
        ],
    )
    def gather_k(idx_hbm, table_hbm, out_hbm, idx_v, rows_v, sem):
        wid = lax.axis_index("s") * nc + lax.axis_index("c")
        pltpu.sync_copy(idx_hbm.at[wid], idx_v)
        handles = [
            pltpu.async_copy(table_hbm.at[idx_v.at[j]],
                             rows_v.at[pl.ds(j * chunk, chunk)], sem)
            for j in range(n_chunks)
        ]
        for h in handles:
            h.wait()
        pltpu.sync_copy(rows_v, out_hbm.at[pl.ds(wid * per_w, per_w)])

    def run(idx_flat, table):
        nonlocal nw, n_chunks
        idx3 = idx_flat.reshape(nw, n_chunks, chunk)
        return gather_k(idx3, table)

    return run


def kernel(h_input, W_in, b_in, W_out, b_out, codebook, W_conv, b_conv):
    m2, cb_pad, cbn = pl.pallas_call(
        _prep_body,
        out_shape=[
            jax.ShapeDtypeStruct((D, 128), jnp.float32),
            jax.ShapeDtypeStruct((K, 128), jnp.float32),
            jax.ShapeDtypeStruct((K, CODE_DIM), jnp.float32),
        ],
    )(W_out, W_conv, b_out.reshape(1, D), b_conv.reshape(1, D), codebook)

    bh = B // 2

    def _argmax_half(off):
        return pl.pallas_call(
            _argmax_body,
            grid=(bh, T // TT),
            in_specs=[
                pl.BlockSpec((1, D, TT), lambda b, t: (b + off, 0, t)),
                pl.BlockSpec((D, CODE_DIM), lambda b, t: (0, 0)),
                pl.BlockSpec((1, CODE_DIM), lambda b, t: (0, 0)),
                pl.BlockSpec((K, CODE_DIM), lambda b, t: (0, 0)),
            ],
            out_specs=pl.BlockSpec((1, 1, TT), lambda b, t: (b, 0, t)),
            out_shape=jax.ShapeDtypeStruct((bh, 1, T), jnp.int32),
            compiler_params=pltpu.CompilerParams(
                dimension_semantics=("parallel", "parallel")),
        )(h_input, W_in, b_in.reshape(1, CODE_DIM), cbn)

    idxa = _argmax_half(0)
    idxb = _argmax_half(bh)

    gather = _make_sc_gather(bh * T, 128)
    c1a = gather(idxa.reshape(bh * T), cb_pad)
    c1b = gather(idxb.reshape(bh * T), cb_pad)

    out_half = pl.pallas_call(
        _out_body,
        grid=(bh, T // TT2),
        in_specs=[
            pl.BlockSpec((1, TT2, 128), lambda b, t: (b, t, 0)),
            pl.BlockSpec((D, 128), lambda b, t: (0, 0)),
        ],
        out_specs=pl.BlockSpec((1, D, TT2), lambda b, t: (b + bh, 0, t)),
        out_shape=jax.ShapeDtypeStruct((B, D, T), jnp.float32),
    )(c1b.reshape(bh, T, 128), m2)

    out = pl.pallas_call(
        _out_body_acc,
        grid=(bh, T // TT2),
        in_specs=[
            pl.BlockSpec((1, TT2, 128), lambda b, t: (b, t, 0)),
            pl.BlockSpec((D, 128), lambda b, t: (0, 0)),
            pl.BlockSpec(memory_space=pl.ANY),
        ],
        out_specs=pl.BlockSpec((1, D, TT2), lambda b, t: (b, 0, t)),
        out_shape=jax.ShapeDtypeStruct((B, D, T), jnp.float32),
        input_output_aliases={2: 0},
    )(c1a.reshape(bh, T, 128), m2, out_half)
    return out
